# Initial kernel scaffold; baseline (speedup 1.0000x reference)
import jax
import jax.numpy as jnp
from jax.experimental import pallas as pl


def kernel(features, alpha, wq, bq, ww, bw):
    raise NotImplementedError("write your pallas kernel here")


# speedup vs baseline: 1.2268x; 1.2268x over previous
import jax
import jax.numpy as jnp
from jax import lax
from jax.experimental import pallas as pl
from jax.experimental.pallas import tpu as pltpu


def _round_up(x, m):
    return ((x + m - 1) // m) * m


def _h_kernel(feat_ref, wqT_ref, bq_ref, h_ref):
    x = feat_ref[...].astype(jnp.bfloat16)
    acc = jnp.dot(x, wqT_ref[...], preferred_element_type=jnp.float32)
    h_ref[...] = jnp.maximum(acc + bq_ref[...], 0.0).astype(jnp.bfloat16)


def _agg_kernel(alpha_ref, h_all_ref, h_dst_ref, w1T_ref, w2T_ref, bw_ref,
                out_ref):
    a16 = alpha_ref[...].astype(jnp.bfloat16)
    hn = jnp.dot(a16, h_all_ref[...], preferred_element_type=jnp.float32)
    z = jnp.dot(h_dst_ref[...], w1T_ref[...],
                preferred_element_type=jnp.float32)
    z = z + jnp.dot(hn.astype(jnp.bfloat16), w2T_ref[...],
                    preferred_element_type=jnp.float32)
    z = jnp.maximum(z + bw_ref[...], 0.0)
    sumsq = jnp.sum(z * z, axis=-1, keepdims=True)
    inv_norm = lax.rsqrt(sumsq + 1e-12)
    out_ref[...] = (z * inv_norm).astype(out_ref.dtype)


def kernel(features, alpha, wq, bq, ww, bw):
    n, in_dim = features.shape
    out_dim = ww.shape[0]
    dtype = features.dtype

    d_pad = _round_up(in_dim, 128)
    o_pad = _round_up(out_dim, 128)
    n_pad = _round_up(n, 128)

    def pad2(x, r, c):
        if x.shape == (r, c):
            return x
        return jnp.pad(x, ((0, r - x.shape[0]), (0, c - x.shape[1])))

    feat_p = pad2(features, n_pad, d_pad)
    alpha_p = pad2(alpha, n_pad, n_pad)
    wqT_p = pad2(wq.T, d_pad, d_pad).astype(jnp.bfloat16)
    bq_p = pad2(bq.reshape(1, in_dim), 1, d_pad)
    w1T_p = pad2(ww[:, :in_dim].T, d_pad, o_pad).astype(jnp.bfloat16)
    w2T_p = pad2(ww[:, in_dim:].T, d_pad, o_pad).astype(jnp.bfloat16)
    bw_p = pad2(bw.reshape(1, out_dim), 1, o_pad)

    vmem_limit = 64 * 1024 * 1024

    tm_h = 512 if n_pad % 512 == 0 else 128
    h = pl.pallas_call(
        _h_kernel,
        out_shape=jax.ShapeDtypeStruct((n_pad, d_pad), jnp.bfloat16),
        grid=(n_pad // tm_h,),
        in_specs=[
            pl.BlockSpec((tm_h, d_pad), lambda i: (i, 0)),
            pl.BlockSpec((d_pad, d_pad), lambda i: (0, 0)),
            pl.BlockSpec((1, d_pad), lambda i: (0, 0)),
        ],
        out_specs=pl.BlockSpec((tm_h, d_pad), lambda i: (i, 0)),
        compiler_params=pltpu.CompilerParams(
            dimension_semantics=("parallel",),
            vmem_limit_bytes=vmem_limit),
    )(feat_p, wqT_p, bq_p)

    tm = 512 if n_pad % 512 == 0 else 128
    out_p = pl.pallas_call(
        _agg_kernel,
        out_shape=jax.ShapeDtypeStruct((n_pad, o_pad), dtype),
        grid=(n_pad // tm,),
        in_specs=[
            pl.BlockSpec((tm, n_pad), lambda i: (i, 0)),
            pl.BlockSpec((n_pad, d_pad), lambda i: (0, 0)),
            pl.BlockSpec((tm, d_pad), lambda i: (i, 0)),
            pl.BlockSpec((d_pad, o_pad), lambda i: (0, 0)),
            pl.BlockSpec((d_pad, o_pad), lambda i: (0, 0)),
            pl.BlockSpec((1, o_pad), lambda i: (0, 0)),
        ],
        out_specs=pl.BlockSpec((tm, o_pad), lambda i: (i, 0)),
        compiler_params=pltpu.CompilerParams(
            dimension_semantics=("parallel",),
            vmem_limit_bytes=vmem_limit),
    )(alpha_p, h, h, w1T_p, w2T_p, bw_p)

    return out_p[:n, :out_dim]


# speedup vs baseline: 1.5212x; 1.2400x over previous
import functools

import jax
import jax.numpy as jnp
from jax import lax
from jax.experimental import pallas as pl
from jax.experimental.pallas import tpu as pltpu


def _round_up(x, m):
    return ((x + m - 1) // m) * m


def _fused_kernel(feat_ref, alpha_ref, wqT_ref, bq_ref, w1T_ref, w2T_ref,
                  bw_ref, out_ref, h_ref, *, nk, tm):
    i0 = pl.program_id(0)
    k = pl.program_id(1)

    @pl.when(k == 0)
    def _():
        x = feat_ref[...].astype(jnp.bfloat16)
        acc = jnp.dot(x, wqT_ref[...], preferred_element_type=jnp.float32)
        h_ref[...] = jnp.maximum(acc + bq_ref[...], 0.0).astype(jnp.bfloat16)

    a16 = alpha_ref[...].astype(jnp.bfloat16)
    hn = jnp.dot(a16, h_ref[...], preferred_element_type=jnp.float32)

    row0 = (i0 * nk + k) * tm
    hd = h_ref[pl.ds(row0, tm), :]
    z = jnp.dot(hd, w1T_ref[...], preferred_element_type=jnp.float32)
    z = z + jnp.dot(hn.astype(jnp.bfloat16), w2T_ref[...],
                    preferred_element_type=jnp.float32)
    z = jnp.maximum(z + bw_ref[...], 0.0)
    sumsq = jnp.sum(z * z, axis=-1, keepdims=True)
    inv_norm = lax.rsqrt(sumsq + 1e-12)
    out_ref[...] = (z * inv_norm).astype(out_ref.dtype)


def kernel(features, alpha, wq, bq, ww, bw):
    n, in_dim = features.shape
    out_dim = ww.shape[0]
    dtype = features.dtype

    d_pad = _round_up(in_dim, 128)
    o_pad = _round_up(out_dim, 128)
    n_pad = _round_up(n, 128)

    def pad2(x, r, c):
        if x.shape == (r, c):
            return x
        return jnp.pad(x, ((0, r - x.shape[0]), (0, c - x.shape[1])))

    feat_p = pad2(features, n_pad, d_pad)
    alpha_p = pad2(alpha, n_pad, n_pad)
    wqT_p = pad2(wq.T, d_pad, d_pad).astype(jnp.bfloat16)
    bq_p = pad2(bq.reshape(1, in_dim), 1, d_pad)
    w1T_p = pad2(ww[:, :in_dim].T, d_pad, o_pad).astype(jnp.bfloat16)
    w2T_p = pad2(ww[:, in_dim:].T, d_pad, o_pad).astype(jnp.bfloat16)
    bw_p = pad2(bw.reshape(1, out_dim), 1, o_pad)

    tm = 512 if n_pad % 1024 == 0 else 128
    nk = n_pad // (2 * tm)

    out_p = pl.pallas_call(
        functools.partial(_fused_kernel, nk=nk, tm=tm),
        out_shape=jax.ShapeDtypeStruct((n_pad, o_pad), dtype),
        grid=(2, nk),
        in_specs=[
            pl.BlockSpec((n_pad, d_pad), lambda i, k: (0, 0)),
            pl.BlockSpec((tm, n_pad), lambda i, k, nk=nk: (i * nk + k, 0)),
            pl.BlockSpec((d_pad, d_pad), lambda i, k: (0, 0)),
            pl.BlockSpec((1, d_pad), lambda i, k: (0, 0)),
            pl.BlockSpec((d_pad, o_pad), lambda i, k: (0, 0)),
            pl.BlockSpec((d_pad, o_pad), lambda i, k: (0, 0)),
            pl.BlockSpec((1, o_pad), lambda i, k: (0, 0)),
        ],
        out_specs=pl.BlockSpec((tm, o_pad), lambda i, k: (i * nk + k, 0)),
        scratch_shapes=[pltpu.VMEM((n_pad, d_pad), jnp.bfloat16)],
        compiler_params=pltpu.CompilerParams(
            dimension_semantics=("parallel", "arbitrary"),
            vmem_limit_bytes=64 * 1024 * 1024),
    )(feat_p, alpha_p, wqT_p, bq_p, w1T_p, w2T_p, bw_p)

    return out_p[:n, :out_dim]
